# baseline (speedup 1.0000x reference)
import functools

import jax
import jax.numpy as jnp
from jax import lax
from jax.experimental import pallas as pl
from jax.experimental.pallas import tpu as pltpu
from jax.experimental.pallas import tpu_sc as plsc

_B = 2
_N = 2048
_T = _B * _N
_DIM = 1024
_DIN = 2048
_H = 4
_DK = 512
_NK = 128
_NE = 16384
_PH = 2 * _H

_RT = 512
_FT = 128


def _fuse_body(wq_ref, k2_ref, m_ref):
    m_ref[...] = lax.dot_general(
        wq_ref[...], k2_ref[0],
        (((0,), (0,)), ((), ())), preferred_element_type=jnp.float32)


def _fuse_call(Wq, K2):
    return pl.pallas_call(
        _fuse_body,
        grid=(_PH,),
        in_specs=[
            pl.BlockSpec((_DK, _DIM), lambda ph: (ph, 0)),
            pl.BlockSpec((1, _DK, _NK), lambda ph: (ph, 0, 0)),
        ],
        out_specs=pl.BlockSpec((_DIM, _NK), lambda ph: (0, ph)),
        out_shape=jax.ShapeDtypeStruct((_DIM, _PH * _NK), jnp.float32),
    )(Wq, K2)


def _router_body(x_ref, m_ref, idx_ref, g_ref):
    sim = lax.dot_general(
        x_ref[...], m_ref[...],
        (((1,), (0,)), ((), ())), preferred_element_type=jnp.float32)
    iota = lax.broadcasted_iota(jnp.int32, (_RT, _NK), 1)
    ms, ams = [], []
    for ph in range(_PH):
        s = sim[:, ph * _NK:(ph + 1) * _NK]
        m = jnp.max(s, axis=1, keepdims=True)
        am = jnp.min(jnp.where(s >= m, iota, _NK), axis=1, keepdims=True)
        ms.append(m)
        ams.append(am)
    gs, ids = [], []
    for h in range(_H):
        sx, ix = ms[h], ams[h]
        sy, iy = ms[_H + h], ams[_H + h]
        gs.append(jnp.maximum(sx + sy, 0.0))
        ids.append(ix * _NK + iy)
    idx_ref[...] = jnp.concatenate(ids, axis=1)
    g_ref[...] = jnp.concatenate(gs, axis=1)


def _router_call(x2, M):
    return pl.pallas_call(
        _router_body,
        grid=(_T // _RT,),
        in_specs=[
            pl.BlockSpec((_RT, _DIM), lambda i: (i, 0)),
            pl.BlockSpec((_DIM, _PH * _NK), lambda i: (0, 0)),
        ],
        out_specs=[
            pl.BlockSpec((_RT, _H), lambda i: (i, 0)),
            pl.BlockSpec((_RT, _H), lambda i: (i, 0)),
        ],
        out_shape=[
            jax.ShapeDtypeStruct((_T, _H), jnp.int32),
            jax.ShapeDtypeStruct((_T, _H), jnp.float32),
        ],
    )(x2, M)


def _sc_gather_call(idx_flat, ta1, tb1, ta2, tb2):
    ne = idx_flat.shape[0]
    info = plsc.get_sparse_core_info()
    nw = info.num_cores * info.num_subcores
    rows_w = ne // nw
    ch = 16
    nch = rows_w // ch
    mesh = plsc.VectorSubcoreMesh(core_axis_name="c", subcore_axis_name="s")

    @functools.partial(
        pl.kernel, mesh=mesh,
        out_type=[
            jax.ShapeDtypeStruct((ne, _DIM), jnp.float32),
            jax.ShapeDtypeStruct((ne, _DIN), jnp.float32),
            jax.ShapeDtypeStruct((ne, _DIN), jnp.float32),
            jax.ShapeDtypeStruct((ne, _DIM), jnp.float32),
        ],
        scratch_types=[
            pltpu.VMEM((rows_w,), jnp.int32),
            pltpu.VMEM((ch, _DIM), jnp.float32),
            pltpu.VMEM((ch, _DIM), jnp.float32),
            pltpu.VMEM((ch, _DIN), jnp.float32),
            pltpu.VMEM((ch, _DIN), jnp.float32),
            pltpu.SemaphoreType.DMA,
            pltpu.SemaphoreType.DMA,
            pltpu.SemaphoreType.DMA,
            pltpu.SemaphoreType.DMA,
            pltpu.SemaphoreType.DMA,
            pltpu.SemaphoreType.DMA,
            pltpu.SemaphoreType.DMA,
            pltpu.SemaphoreType.DMA,
        ])
    def gk(idx_hbm, a1_hbm, b1_hbm, a2_hbm, b2_hbm,
           oa1, ob1, oa2, ob2,
           idx_v, bn0, bn1, bw0, bw1,
           gs0, gs1, gs2, gs3, ws0, ws1, ws2, ws3):
        wid = lax.axis_index("s") * info.num_cores + lax.axis_index("c")
        base = wid * rows_w
        pltpu.sync_copy(idx_hbm.at[pl.ds(base, rows_w)], idx_v)

        def mk_ops(tab, out):
            def sg(c, buf, sem):
                pltpu.async_copy(tab.at[idx_v.at[pl.ds(c * ch, ch)]], buf, sem)

            def wg(buf, sem):
                pltpu.make_async_copy(
                    tab.at[idx_v.at[pl.ds(0, ch)]], buf, sem).wait()

            def sw(c, buf, sem):
                pltpu.async_copy(buf, out.at[pl.ds(base + c * ch, ch)], sem)

            def ww(buf, sem):
                pltpu.make_async_copy(
                    buf, out.at[pl.ds(base, ch)], sem).wait()

            return sg, wg, sw, ww

        def do_pair(tabA, outA, bA0, bA1, tabB, outB, bB0, bB1):
            sgA, wgA, swA, wwA = mk_ops(tabA, outA)
            sgB, wgB, swB, wwB = mk_ops(tabB, outB)
            sgA(0, bA0, gs0)
            sgB(0, bB0, gs2)

            def body(i, carry):
                c0 = 2 * i
                c1 = c0 + 1
                wgA(bA0, gs0)

                @pl.when(i > 0)
                def _():
                    wwA(bA1, ws1)
                sgA(c1, bA1, gs1)
                swA(c0, bA0, ws0)

                wgB(bB0, gs2)

                @pl.when(i > 0)
                def _():
                    wwB(bB1, ws3)
                sgB(c1, bB1, gs3)
                swB(c0, bB0, ws2)

                wgA(bA1, gs1)
                wwA(bA0, ws0)

                @pl.when(i < nch // 2 - 1)
                def _():
                    sgA(c0 + 2, bA0, gs0)
                swA(c1, bA1, ws1)

                wgB(bB1, gs3)
                wwB(bB0, ws2)

                @pl.when(i < nch // 2 - 1)
                def _():
                    sgB(c0 + 2, bB0, gs2)
                swB(c1, bB1, ws3)
                return carry

            lax.fori_loop(0, nch // 2, body, 0)
            wwA(bA1, ws1)
            wwB(bB1, ws3)

        do_pair(a1_hbm, oa1, bn0, bn1, b1_hbm, ob1, bw0, bw1)
        do_pair(a2_hbm, oa2, bw0, bw1, b2_hbm, ob2, bn0, bn1)

    return gk(idx_flat, ta1, tb1, ta2, tb2)


def _ffn_body(x_ref, g_ref, ga1_ref, gb1_ref, ga2_ref, gb2_ref,
              win_ref, wout_ref, out_ref):
    x = x_ref[...]
    g = g_ref[...]
    hid = lax.dot_general(
        x, win_ref[...], (((1,), (1,)), ((), ())),
        preferred_element_type=jnp.float32)
    acc = jnp.zeros((_FT, _DIN), jnp.float32)
    for h in range(_H):
        s = jnp.sum(x * ga1_ref[h], axis=1, keepdims=True)
        c = g[:, h:h + 1] * s
        acc = acc + c * gb1_ref[h]
    hid = hid + acc
    hid = 0.5 * hid * (1.0 + lax.erf(hid * 0.7071067811865476))
    outv = lax.dot_general(
        hid, wout_ref[...], (((1,), (1,)), ((), ())),
        preferred_element_type=jnp.float32)
    acc2 = jnp.zeros((_FT, _DIM), jnp.float32)
    for h in range(_H):
        s2 = jnp.sum(hid * ga2_ref[h], axis=1, keepdims=True)
        c2 = g[:, h:h + 1] * s2
        acc2 = acc2 + c2 * gb2_ref[h]
    out_ref[...] = outv + acc2


def _ffn_call(x2, g, ga1, gb1, ga2, gb2, W_in, W_out):
    t = x2.shape[0]
    return pl.pallas_call(
        _ffn_body,
        grid=(t // _FT,),
        in_specs=[
            pl.BlockSpec((_FT, _DIM), lambda i: (i, 0)),
            pl.BlockSpec((_FT, _H), lambda i: (i, 0)),
            pl.BlockSpec((_H, _FT, _DIM), lambda i: (0, i, 0)),
            pl.BlockSpec((_H, _FT, _DIN), lambda i: (0, i, 0)),
            pl.BlockSpec((_H, _FT, _DIN), lambda i: (0, i, 0)),
            pl.BlockSpec((_H, _FT, _DIM), lambda i: (0, i, 0)),
            pl.BlockSpec((_DIN, _DIM), lambda i: (0, 0)),
            pl.BlockSpec((_DIM, _DIN), lambda i: (0, 0)),
        ],
        out_specs=pl.BlockSpec((_FT, _DIM), lambda i: (i, 0)),
        out_shape=jax.ShapeDtypeStruct((t, _DIM), jnp.float32),
    )(x2, g, ga1, gb1, ga2, gb2, W_in, W_out)


def kernel(x, Wq, keys, W_in, W_out, in_lora_a, in_lora_b,
           out_lora_a, out_lora_b):
    b, n, d = x.shape
    x2 = x.reshape(b * n, d)
    K2 = jnp.transpose(keys, (2, 0, 3, 1)).reshape(_PH, _DK, _NK)
    M = _fuse_call(Wq, K2)
    idx, g = _router_call(x2, M)
    nsplit = 2
    th = _T // nsplit
    outs = []
    for k in range(nsplit):
        idx_k = jnp.transpose(idx[k * th:(k + 1) * th]).reshape(-1)
        ga1, gb1, ga2, gb2 = _sc_gather_call(
            idx_k, in_lora_a, in_lora_b, out_lora_a, out_lora_b)
        outs.append(_ffn_call(
            x2[k * th:(k + 1) * th], g[k * th:(k + 1) * th],
            ga1.reshape(_H, th, _DIM), gb1.reshape(_H, th, _DIN),
            ga2.reshape(_H, th, _DIN), gb2.reshape(_H, th, _DIM),
            W_in, W_out))
    out2 = jnp.concatenate(outs, axis=0)
    return out2.reshape(b, n, d)


# speedup vs baseline: 1.0625x; 1.0625x over previous
import functools

import jax
import jax.numpy as jnp
from jax import lax
from jax.experimental import pallas as pl
from jax.experimental.pallas import tpu as pltpu
from jax.experimental.pallas import tpu_sc as plsc

_B = 2
_N = 2048
_T = _B * _N
_DIM = 1024
_DIN = 2048
_H = 4
_DK = 512
_NK = 128
_NE = 16384
_PH = 2 * _H

_RT = 512
_FT = 128


def _fuse_body(wq_ref, k2_ref, m_ref):
    m_ref[...] = lax.dot_general(
        wq_ref[...], k2_ref[0],
        (((0,), (0,)), ((), ())), preferred_element_type=jnp.float32)


def _fuse_call(Wq, K2):
    return pl.pallas_call(
        _fuse_body,
        grid=(_PH,),
        in_specs=[
            pl.BlockSpec((_DK, _DIM), lambda ph: (ph, 0)),
            pl.BlockSpec((1, _DK, _NK), lambda ph: (ph, 0, 0)),
        ],
        out_specs=pl.BlockSpec((_DIM, _NK), lambda ph: (0, ph)),
        out_shape=jax.ShapeDtypeStruct((_DIM, _PH * _NK), jnp.float32),
    )(Wq, K2)


def _router_body(x_ref, m_ref, idx_ref, g_ref):
    sim = lax.dot_general(
        x_ref[...], m_ref[...],
        (((1,), (0,)), ((), ())), preferred_element_type=jnp.float32)
    iota = lax.broadcasted_iota(jnp.int32, (_RT, _NK), 1)
    ms, ams = [], []
    for ph in range(_PH):
        s = sim[:, ph * _NK:(ph + 1) * _NK]
        m = jnp.max(s, axis=1, keepdims=True)
        am = jnp.min(jnp.where(s >= m, iota, _NK), axis=1, keepdims=True)
        ms.append(m)
        ams.append(am)
    gs, ids = [], []
    for h in range(_H):
        sx, ix = ms[h], ams[h]
        sy, iy = ms[_H + h], ams[_H + h]
        gs.append(jnp.maximum(sx + sy, 0.0))
        ids.append(ix * _NK + iy)
    idx_ref[...] = jnp.concatenate(ids, axis=1)
    g_ref[...] = jnp.concatenate(gs, axis=1)


def _router_call(x2, M):
    return pl.pallas_call(
        _router_body,
        grid=(_T // _RT,),
        in_specs=[
            pl.BlockSpec((_RT, _DIM), lambda i: (i, 0)),
            pl.BlockSpec((_DIM, _PH * _NK), lambda i: (0, 0)),
        ],
        out_specs=[
            pl.BlockSpec((_RT, _H), lambda i: (i, 0)),
            pl.BlockSpec((_RT, _H), lambda i: (i, 0)),
        ],
        out_shape=[
            jax.ShapeDtypeStruct((_T, _H), jnp.int32),
            jax.ShapeDtypeStruct((_T, _H), jnp.float32),
        ],
    )(x2, M)


def _sc_gather_call(idx_flat, ta1, tb1, ta2, tb2):
    ne = idx_flat.shape[0]
    info = plsc.get_sparse_core_info()
    nw = info.num_cores * info.num_subcores
    rows_w = ne // nw
    ch = 8
    nch = rows_w // ch
    mesh = plsc.VectorSubcoreMesh(core_axis_name="c", subcore_axis_name="s")

    @functools.partial(
        pl.kernel, mesh=mesh,
        out_type=[
            jax.ShapeDtypeStruct((ne, _DIM), jnp.float32),
            jax.ShapeDtypeStruct((ne, _DIN), jnp.float32),
            jax.ShapeDtypeStruct((ne, _DIN), jnp.float32),
            jax.ShapeDtypeStruct((ne, _DIM), jnp.float32),
        ],
        scratch_types=[
            pltpu.VMEM((rows_w,), jnp.int32),
            pltpu.VMEM((ch, _DIM), jnp.float32),
            pltpu.VMEM((ch, _DIM), jnp.float32),
            pltpu.VMEM((ch, _DIN), jnp.float32),
            pltpu.VMEM((ch, _DIN), jnp.float32),
            pltpu.VMEM((ch, _DIN), jnp.float32),
            pltpu.VMEM((ch, _DIN), jnp.float32),
            pltpu.VMEM((ch, _DIM), jnp.float32),
            pltpu.VMEM((ch, _DIM), jnp.float32),
        ] + [pltpu.SemaphoreType.DMA] * 16)
    def gk(idx_hbm, a1_hbm, b1_hbm, a2_hbm, b2_hbm,
           oa1, ob1, oa2, ob2,
           idx_v, ba0, ba1, bb0, bb1, bc0, bc1, bd0, bd1,
           g0, g1, g2, g3, g4, g5, g6, g7,
           w0, w1, w2, w3, w4, w5, w6, w7):
        wid = lax.axis_index("s") * info.num_cores + lax.axis_index("c")
        base = wid * rows_w
        pltpu.sync_copy(idx_hbm.at[pl.ds(base, rows_w)], idx_v)

        tabs = [
            (a1_hbm, oa1, ba0, ba1, g0, g1, w0, w1),
            (b1_hbm, ob1, bb0, bb1, g2, g3, w2, w3),
            (a2_hbm, oa2, bc0, bc1, g4, g5, w4, w5),
            (b2_hbm, ob2, bd0, bd1, g6, g7, w6, w7),
        ]

        def mk_ops(tab, out):
            def sg(c, buf, sem):
                pltpu.async_copy(tab.at[idx_v.at[pl.ds(c * ch, ch)]], buf, sem)

            def wg(buf, sem):
                pltpu.make_async_copy(
                    tab.at[idx_v.at[pl.ds(0, ch)]], buf, sem).wait()

            def sw(c, buf, sem):
                pltpu.async_copy(buf, out.at[pl.ds(base + c * ch, ch)], sem)

            def ww(buf, sem):
                pltpu.make_async_copy(
                    buf, out.at[pl.ds(base, ch)], sem).wait()

            return sg, wg, sw, ww

        ops = [mk_ops(t[0], t[1]) for t in tabs]
        for (sg, _, _, _), t in zip(ops, tabs):
            sg(0, t[2], t[4])

        def body(i, carry):
            c0 = 2 * i
            c1 = c0 + 1
            for (sg, wg, sw, ww), (_, _, b0, b1, gsa, gsb, wsa, wsb) in zip(
                    ops, tabs):
                wg(b0, gsa)

                @pl.when(i > 0)
                def _():
                    ww(b1, wsb)
                sg(c1, b1, gsb)
                sw(c0, b0, wsa)
            for (sg, wg, sw, ww), (_, _, b0, b1, gsa, gsb, wsa, wsb) in zip(
                    ops, tabs):
                wg(b1, gsb)
                ww(b0, wsa)

                @pl.when(i < nch // 2 - 1)
                def _():
                    sg(c0 + 2, b0, gsa)
                sw(c1, b1, wsb)
            return carry

        lax.fori_loop(0, nch // 2, body, 0)
        for (_, _, _, ww), (_, _, b0, b1, gsa, gsb, wsa, wsb) in zip(
                ops, tabs):
            ww(b1, wsb)

    return gk(idx_flat, ta1, tb1, ta2, tb2)


def _ffn_body(x_ref, g_ref, ga1_ref, gb1_ref, ga2_ref, gb2_ref,
              win_ref, wout_ref, out_ref):
    x = x_ref[...]
    g = g_ref[...]
    hid = lax.dot_general(
        x, win_ref[...], (((1,), (1,)), ((), ())),
        preferred_element_type=jnp.float32)
    acc = jnp.zeros((_FT, _DIN), jnp.float32)
    for h in range(_H):
        s = jnp.sum(x * ga1_ref[h], axis=1, keepdims=True)
        c = g[:, h:h + 1] * s
        acc = acc + c * gb1_ref[h]
    hid = hid + acc
    hid = 0.5 * hid * (1.0 + lax.erf(hid * 0.7071067811865476))
    outv = lax.dot_general(
        hid, wout_ref[...], (((1,), (1,)), ((), ())),
        preferred_element_type=jnp.float32)
    acc2 = jnp.zeros((_FT, _DIM), jnp.float32)
    for h in range(_H):
        s2 = jnp.sum(hid * ga2_ref[h], axis=1, keepdims=True)
        c2 = g[:, h:h + 1] * s2
        acc2 = acc2 + c2 * gb2_ref[h]
    out_ref[...] = outv + acc2


def _ffn_call(x2, g, ga1, gb1, ga2, gb2, W_in, W_out):
    t = x2.shape[0]
    return pl.pallas_call(
        _ffn_body,
        grid=(t // _FT,),
        in_specs=[
            pl.BlockSpec((_FT, _DIM), lambda i: (i, 0)),
            pl.BlockSpec((_FT, _H), lambda i: (i, 0)),
            pl.BlockSpec((_H, _FT, _DIM), lambda i: (0, i, 0)),
            pl.BlockSpec((_H, _FT, _DIN), lambda i: (0, i, 0)),
            pl.BlockSpec((_H, _FT, _DIN), lambda i: (0, i, 0)),
            pl.BlockSpec((_H, _FT, _DIM), lambda i: (0, i, 0)),
            pl.BlockSpec((_DIN, _DIM), lambda i: (0, 0)),
            pl.BlockSpec((_DIM, _DIN), lambda i: (0, 0)),
        ],
        out_specs=pl.BlockSpec((_FT, _DIM), lambda i: (i, 0)),
        out_shape=jax.ShapeDtypeStruct((t, _DIM), jnp.float32),
    )(x2, g, ga1, gb1, ga2, gb2, W_in, W_out)


def kernel(x, Wq, keys, W_in, W_out, in_lora_a, in_lora_b,
           out_lora_a, out_lora_b):
    b, n, d = x.shape
    x2 = x.reshape(b * n, d)
    K2 = jnp.transpose(keys, (2, 0, 3, 1)).reshape(_PH, _DK, _NK)
    M = _fuse_call(Wq, K2)
    idx, g = _router_call(x2, M)
    idx_flat = jnp.transpose(idx).reshape(-1)
    ga1, gb1, ga2, gb2 = _sc_gather_call(
        idx_flat, in_lora_a, in_lora_b, out_lora_a, out_lora_b)
    out2 = _ffn_call(
        x2, g,
        ga1.reshape(_H, _T, _DIM), gb1.reshape(_H, _T, _DIN),
        ga2.reshape(_H, _T, _DIN), gb2.reshape(_H, _T, _DIM),
        W_in, W_out)
    return out2.reshape(b, n, d)


# speedup vs baseline: 1.0681x; 1.0053x over previous
import functools

import jax
import jax.numpy as jnp
from jax import lax
from jax.experimental import pallas as pl
from jax.experimental.pallas import tpu as pltpu
from jax.experimental.pallas import tpu_sc as plsc

_B = 2
_N = 2048
_T = _B * _N
_DIM = 1024
_DIN = 2048
_H = 4
_DK = 512
_NK = 128
_NE = 16384
_PH = 2 * _H

_RT = 512
_FT = 128


def _fuse_body(wq_ref, k2_ref, m_ref):
    m_ref[...] = lax.dot_general(
        wq_ref[...], k2_ref[0],
        (((0,), (0,)), ((), ())), preferred_element_type=jnp.float32)


def _fuse_call(Wq, K2):
    return pl.pallas_call(
        _fuse_body,
        grid=(_PH,),
        in_specs=[
            pl.BlockSpec((_DK, _DIM), lambda ph: (ph, 0)),
            pl.BlockSpec((1, _DK, _NK), lambda ph: (ph, 0, 0)),
        ],
        out_specs=pl.BlockSpec((_DIM, _NK), lambda ph: (0, ph)),
        out_shape=jax.ShapeDtypeStruct((_DIM, _PH * _NK), jnp.float32),
    )(Wq, K2)


def _router_body(x_ref, m_ref, idx_ref, g_ref):
    sim = lax.dot_general(
        x_ref[...], m_ref[...],
        (((1,), (0,)), ((), ())), preferred_element_type=jnp.float32)
    iota = lax.broadcasted_iota(jnp.int32, (_RT, _NK), 1)
    ms, ams = [], []
    for ph in range(_PH):
        s = sim[:, ph * _NK:(ph + 1) * _NK]
        m = jnp.max(s, axis=1, keepdims=True)
        am = jnp.min(jnp.where(s >= m, iota, _NK), axis=1, keepdims=True)
        ms.append(m)
        ams.append(am)
    gs, ids = [], []
    for h in range(_H):
        sx, ix = ms[h], ams[h]
        sy, iy = ms[_H + h], ams[_H + h]
        gs.append(jnp.maximum(sx + sy, 0.0))
        ids.append(ix * _NK + iy)
    idx_ref[...] = jnp.concatenate(ids, axis=1)
    g_ref[...] = jnp.concatenate(gs, axis=1)


def _router_call(x2, M):
    return pl.pallas_call(
        _router_body,
        grid=(_T // _RT,),
        in_specs=[
            pl.BlockSpec((_RT, _DIM), lambda i: (i, 0)),
            pl.BlockSpec((_DIM, _PH * _NK), lambda i: (0, 0)),
        ],
        out_specs=[
            pl.BlockSpec((_RT, _H), lambda i: (i, 0)),
            pl.BlockSpec((_RT, _H), lambda i: (i, 0)),
        ],
        out_shape=[
            jax.ShapeDtypeStruct((_T, _H), jnp.int32),
            jax.ShapeDtypeStruct((_T, _H), jnp.float32),
        ],
    )(x2, M)


def _sc_gather_call(idx_flat, x2, ta1, tb1, ta2, tb2):
    ne = idx_flat.shape[0]
    info = plsc.get_sparse_core_info()
    nw = info.num_cores * info.num_subcores
    rows_w = ne // nw
    ch = 8
    nch = rows_w // ch
    nlane = _DIM // 16
    mesh = plsc.VectorSubcoreMesh(core_axis_name="c", subcore_axis_name="s")

    @functools.partial(
        pl.kernel, mesh=mesh,
        out_type=[
            jax.ShapeDtypeStruct((ne * 16,), jnp.float32),
            jax.ShapeDtypeStruct((ne, _DIN), jnp.float32),
            jax.ShapeDtypeStruct((ne, _DIN), jnp.float32),
            jax.ShapeDtypeStruct((ne, _DIM), jnp.float32),
        ],
        scratch_types=[
            pltpu.VMEM((rows_w,), jnp.int32),
            pltpu.VMEM((ch, _DIM), jnp.float32),
            pltpu.VMEM((ch, _DIM), jnp.float32),
            pltpu.VMEM((ch, _DIM), jnp.float32),
            pltpu.VMEM((ch, _DIM), jnp.float32),
            pltpu.VMEM((ch, _DIN), jnp.float32),
            pltpu.VMEM((ch, _DIN), jnp.float32),
            pltpu.VMEM((ch, _DIN), jnp.float32),
            pltpu.VMEM((ch, _DIN), jnp.float32),
            pltpu.VMEM((ch, _DIM), jnp.float32),
            pltpu.VMEM((ch, _DIM), jnp.float32),
            pltpu.VMEM((rows_w * 16,), jnp.float32),
        ] + [pltpu.SemaphoreType.DMA] * 16)
    def gk(idx_hbm, x_hbm, a1_hbm, b1_hbm, a2_hbm, b2_hbm,
           ocp, ob1, oa2, ob2,
           idx_v, ba0, ba1, bx0, bx1, bb0, bb1, bc0, bc1, bd0, bd1, cp_v,
           ga0, ga1s, gx0, gx1, g2, g3, g4, g5, g6, g7,
           w2, w3, w4, w5, w6, w7):
        wid = lax.axis_index("s") * info.num_cores + lax.axis_index("c")
        base = wid * rows_w
        tbase = lax.rem(base, _T)
        pltpu.sync_copy(idx_hbm.at[pl.ds(base, rows_w)], idx_v)

        tabs = [
            (b1_hbm, ob1, bb0, bb1, g2, g3, w2, w3),
            (a2_hbm, oa2, bc0, bc1, g4, g5, w4, w5),
            (b2_hbm, ob2, bd0, bd1, g6, g7, w6, w7),
        ]

        def mk_ops(tab, out):
            def sg(c, buf, sem):
                pltpu.async_copy(tab.at[idx_v.at[pl.ds(c * ch, ch)]], buf, sem)

            def wg(buf, sem):
                pltpu.make_async_copy(
                    tab.at[idx_v.at[pl.ds(0, ch)]], buf, sem).wait()

            def sw(c, buf, sem):
                pltpu.async_copy(buf, out.at[pl.ds(base + c * ch, ch)], sem)

            def ww(buf, sem):
                pltpu.make_async_copy(
                    buf, out.at[pl.ds(base, ch)], sem).wait()

            return sg, wg, sw, ww

        ops = [mk_ops(t[0], t[1]) for t in tabs]

        def sga(c, buf, sem):
            pltpu.async_copy(a1_hbm.at[idx_v.at[pl.ds(c * ch, ch)]], buf, sem)

        def wga(buf, sem):
            pltpu.make_async_copy(
                a1_hbm.at[idx_v.at[pl.ds(0, ch)]], buf, sem).wait()

        def sgx(c, buf, sem):
            pltpu.async_copy(x_hbm.at[pl.ds(tbase + c * ch, ch)], buf, sem)

        def wgx(buf, sem):
            pltpu.make_async_copy(
                x_hbm.at[pl.ds(tbase, ch)], buf, sem).wait()

        def compute(c, abuf, xbuf):
            for r in range(ch):
                accs = [jnp.zeros((16,), jnp.float32) for _ in range(4)]
                for j in range(nlane):
                    accs[j % 4] = accs[j % 4] + (
                        abuf[r, pl.ds(j * 16, 16)] * xbuf[r, pl.ds(j * 16, 16)])
                acc = (accs[0] + accs[1]) + (accs[2] + accs[3])
                cp_v[pl.ds((c * ch + r) * 16, 16)] = acc

        for (sg, _, _, _), t in zip(ops, tabs):
            sg(0, t[2], t[4])
        sga(0, ba0, ga0)
        sgx(0, bx0, gx0)

        def body(i, carry):
            c0 = 2 * i
            c1 = c0 + 1
            for (sg, wg, sw, ww), (_, _, b0, b1, gsa, gsb, wsa, wsb) in zip(
                    ops, tabs):
                wg(b0, gsa)

                @pl.when(i > 0)
                def _():
                    ww(b1, wsb)
                sg(c1, b1, gsb)
                sw(c0, b0, wsa)
            wga(ba0, ga0)
            wgx(bx0, gx0)
            sga(c1, ba1, ga1s)
            sgx(c1, bx1, gx1)
            compute(c0, ba0, bx0)
            for (sg, wg, sw, ww), (_, _, b0, b1, gsa, gsb, wsa, wsb) in zip(
                    ops, tabs):
                wg(b1, gsb)
                ww(b0, wsa)

                @pl.when(i < nch // 2 - 1)
                def _():
                    sg(c0 + 2, b0, gsa)
                sw(c1, b1, wsb)
            wga(ba1, ga1s)
            wgx(bx1, gx1)

            @pl.when(i < nch // 2 - 1)
            def _():
                sga(c0 + 2, ba0, ga0)
                sgx(c0 + 2, bx0, gx0)
            compute(c1, ba1, bx1)
            return carry

        lax.fori_loop(0, nch // 2, body, 0)
        for (_, _, _, ww), (_, _, b0, b1, gsa, gsb, wsa, wsb) in zip(
                ops, tabs):
            ww(b1, wsb)
        pltpu.sync_copy(cp_v, ocp.at[pl.ds(base * 16, rows_w * 16)])

    return gk(idx_flat, x2, ta1, tb1, ta2, tb2)


def _ffn_body(x_ref, g_ref, cp_ref, gb1_ref, ga2_ref, gb2_ref,
              win_ref, wout_ref, out_ref):
    x = x_ref[...]
    g = g_ref[...]
    hid = lax.dot_general(
        x, win_ref[...], (((1,), (1,)), ((), ())),
        preferred_element_type=jnp.float32)
    acc = jnp.zeros((_FT, _DIN), jnp.float32)
    for h in range(_H):
        s = jnp.sum(cp_ref[h], axis=1, keepdims=True)
        c = g[:, h:h + 1] * s
        acc = acc + c * gb1_ref[h]
    hid = hid + acc
    hid = 0.5 * hid * (1.0 + lax.erf(hid * 0.7071067811865476))
    outv = lax.dot_general(
        hid, wout_ref[...], (((1,), (1,)), ((), ())),
        preferred_element_type=jnp.float32)
    acc2 = jnp.zeros((_FT, _DIM), jnp.float32)
    for h in range(_H):
        s2 = jnp.sum(hid * ga2_ref[h], axis=1, keepdims=True)
        c2 = g[:, h:h + 1] * s2
        acc2 = acc2 + c2 * gb2_ref[h]
    out_ref[...] = outv + acc2


def _ffn_call(x2, g, cp, gb1, ga2, gb2, W_in, W_out):
    t = x2.shape[0]
    return pl.pallas_call(
        _ffn_body,
        grid=(t // _FT,),
        in_specs=[
            pl.BlockSpec((_FT, _DIM), lambda i: (i, 0)),
            pl.BlockSpec((_FT, _H), lambda i: (i, 0)),
            pl.BlockSpec((_H, _FT, 16), lambda i: (0, i, 0)),
            pl.BlockSpec((_H, _FT, _DIN), lambda i: (0, i, 0)),
            pl.BlockSpec((_H, _FT, _DIN), lambda i: (0, i, 0)),
            pl.BlockSpec((_H, _FT, _DIM), lambda i: (0, i, 0)),
            pl.BlockSpec((_DIN, _DIM), lambda i: (0, 0)),
            pl.BlockSpec((_DIM, _DIN), lambda i: (0, 0)),
        ],
        out_specs=pl.BlockSpec((_FT, _DIM), lambda i: (i, 0)),
        out_shape=jax.ShapeDtypeStruct((t, _DIM), jnp.float32),
    )(x2, g, cp, gb1, ga2, gb2, W_in, W_out)


def kernel(x, Wq, keys, W_in, W_out, in_lora_a, in_lora_b,
           out_lora_a, out_lora_b):
    b, n, d = x.shape
    x2 = x.reshape(b * n, d)
    K2 = jnp.transpose(keys, (2, 0, 3, 1)).reshape(_PH, _DK, _NK)
    M = _fuse_call(Wq, K2)
    idx, g = _router_call(x2, M)
    idx_flat = jnp.transpose(idx).reshape(-1)
    cp, gb1, ga2, gb2 = _sc_gather_call(
        idx_flat, x2, in_lora_a, in_lora_b, out_lora_a, out_lora_b)
    out2 = _ffn_call(
        x2, g,
        cp.reshape(_H, _T, 16), gb1.reshape(_H, _T, _DIN),
        ga2.reshape(_H, _T, _DIN), gb2.reshape(_H, _T, _DIM),
        W_in, W_out)
    return out2.reshape(b, n, d)
